# baseline (speedup 1.0000x reference)
import functools

import jax
import jax.numpy as jnp
from jax import lax
from jax.experimental import pallas as pl
from jax.experimental.pallas import tpu as pltpu
from jax.experimental.pallas import tpu_sc as plsc

N = 10000
E = 320000
D = 128
NW = 32
EPW = E // NW
CPW = 2 * E // NW
GB = 80
NBG = CPW // GB
NBS = EPW // GB
SB = 2000
RPT = 640
RC = 80
TE = 2560
TN = 2000
NORM_FACTOR = 100.0

_mesh = plsc.VectorSubcoreMesh(core_axis_name="c", subcore_axis_name="s")
_sc_params = pltpu.CompilerParams(needs_layout_passes=False)


def _worker_id():
    return lax.axis_index("s") * 2 + lax.axis_index("c")


def _pipelined_gather(h_hbm, idxcat_hbm, hcat_hbm, idx_all, bufs, sgs, sws,
                      base):
    pltpu.sync_copy(idxcat_hbm.at[pl.ds(base, CPW)], idx_all)
    pltpu.async_copy(h_hbm.at[idx_all.at[pl.ds(0, GB)]], bufs[0], sgs[0])

    @pl.loop(0, NBG, step=2)
    def _loop(g0):
        for b in range(2):
            g = g0 + b
            buf, sg, sw = bufs[b], sgs[b], sws[b]
            obuf, osg, osw = bufs[1 - b], sgs[1 - b], sws[1 - b]
            pltpu.make_async_copy(
                h_hbm.at[idx_all.at[pl.ds(g * GB, GB)]], buf, sg).wait()
            pltpu.async_copy(buf, hcat_hbm.at[pl.ds(base + g * GB, GB)], sw)

            @pl.when(g >= 1)
            def _():
                pltpu.make_async_copy(
                    obuf, hcat_hbm.at[pl.ds(base + (g - 1) * GB, GB)],
                    osw).wait()

            @pl.when(g + 1 < NBG)
            def _():
                pltpu.async_copy(
                    h_hbm.at[idx_all.at[pl.ds((g + 1) * GB, GB)]], obuf, osg)

    last = (NBG - 1) % 2
    pltpu.make_async_copy(
        bufs[last], hcat_hbm.at[pl.ds(base + (NBG - 1) * GB, GB)],
        sws[last]).wait()


@functools.partial(
    pl.kernel, mesh=_mesh,
    out_type=[
        jax.ShapeDtypeStruct((2 * E, D), jnp.float32),
        jax.ShapeDtypeStruct((4 * E,), jnp.float32),
    ],
    scratch_types=[
        pltpu.VMEM((CPW,), jnp.int32),
        pltpu.VMEM((EPW,), jnp.int32),
        pltpu.VMEM((EPW,), jnp.int32),
        pltpu.VMEM((GB, D), jnp.float32),
        pltpu.VMEM((GB, D), jnp.float32),
        pltpu.VMEM((N,), jnp.float32),
        pltpu.VMEM((N,), jnp.float32),
        pltpu.VMEM((N,), jnp.float32),
        pltpu.VMEM((SB,), jnp.float32),
        pltpu.VMEM((SB,), jnp.float32),
        pltpu.VMEM((SB,), jnp.float32),
        pltpu.VMEM((SB,), jnp.float32),
        pltpu.SemaphoreType.DMA,
        pltpu.SemaphoreType.DMA,
        pltpu.SemaphoreType.DMA,
        pltpu.SemaphoreType.DMA,
    ],
    compiler_params=_sc_params,
)
def _gather_hx(h_hbm, x0_hbm, x1_hbm, x2_hbm, idxcat_hbm,
               hcat_hbm, flat4_hbm,
               idx_all, idx_r, idx_c, b0, b1, x0, x1, x2,
               sr, s0, s1, s2, sg0, sg1, sw0, sw1):
    wid = _worker_id()
    pbase = wid * EPW
    pltpu.sync_copy(idxcat_hbm.at[pl.ds(pbase, EPW)], idx_r)
    pltpu.sync_copy(idxcat_hbm.at[pl.ds(E + pbase, EPW)], idx_c)
    pltpu.sync_copy(x0_hbm, x0)
    pltpu.sync_copy(x1_hbm, x1)
    pltpu.sync_copy(x2_hbm, x2)

    def chunk(c, carry):
        def group(k, c2):
            src = pl.ds(c * SB + k * 16, 16)
            dst = pl.ds(k * 16, 16)
            ir = idx_r[src]
            ic = idx_c[src]
            d0 = plsc.load_gather(x0, [ir]) - plsc.load_gather(x0, [ic])
            d1 = plsc.load_gather(x1, [ir]) - plsc.load_gather(x1, [ic])
            d2 = plsc.load_gather(x2, [ir]) - plsc.load_gather(x2, [ic])
            sr[dst] = d0 * d0 + d1 * d1 + d2 * d2
            s0[dst] = d0
            s1[dst] = d1
            s2[dst] = d2
            return c2
        lax.fori_loop(0, SB // 16, group, carry)
        off = pbase + c * SB
        pltpu.sync_copy(sr, flat4_hbm.at[pl.ds(0 * E + off, SB)])
        pltpu.sync_copy(s0, flat4_hbm.at[pl.ds(1 * E + off, SB)])
        pltpu.sync_copy(s1, flat4_hbm.at[pl.ds(2 * E + off, SB)])
        pltpu.sync_copy(s2, flat4_hbm.at[pl.ds(3 * E + off, SB)])
        return carry
    lax.fori_loop(0, EPW // SB, chunk, 0)

    _pipelined_gather(h_hbm, idxcat_hbm, hcat_hbm, idx_all,
                      (b0, b1), (sg0, sg1), (sw0, sw1), wid * CPW)


@functools.partial(
    pl.kernel, mesh=_mesh,
    out_type=jax.ShapeDtypeStruct((2 * E, D), jnp.float32),
    scratch_types=[
        pltpu.VMEM((CPW,), jnp.int32),
        pltpu.VMEM((GB, D), jnp.float32),
        pltpu.VMEM((GB, D), jnp.float32),
        pltpu.SemaphoreType.DMA,
        pltpu.SemaphoreType.DMA,
        pltpu.SemaphoreType.DMA,
        pltpu.SemaphoreType.DMA,
    ],
    compiler_params=_sc_params,
)
def _gather_h(h_hbm, idxcat_hbm, hcat_hbm,
              idx_all, b0, b1, sg0, sg1, sw0, sw1):
    _pipelined_gather(h_hbm, idxcat_hbm, hcat_hbm, idx_all,
                      (b0, b1), (sg0, sg1), (sw0, sw1), _worker_id() * CPW)


@functools.partial(
    pl.kernel, mesh=_mesh,
    out_type=jax.ShapeDtypeStruct((2 * N, D), jnp.float32),
    scratch_types=[
        pltpu.VMEM((GB,), jnp.int32),
        pltpu.VMEM((GB,), jnp.int32),
        pltpu.VMEM((GB, D), jnp.float32),
        pltpu.VMEM((GB, D), jnp.float32),
        pltpu.VMEM((RC, D), jnp.float32),
        pltpu.VMEM_SHARED((N, D), jnp.float32),
        pltpu.SemaphoreType.DMA,
        pltpu.SemaphoreType.DMA,
        pltpu.SemaphoreType.DMA,
        pltpu.SemaphoreType.DMA,
        pltpu.SemaphoreType.DMA,
        pltpu.SemaphoreType.DMA,
    ],
    compiler_params=_sc_params,
)
def _scatter_d(feat_hbm, row_hbm, out_hbm,
               i0, i1, f0, f1, stage, table,
               si0, si1, sf0, sf1, sa0, sa1):
    cid = lax.axis_index("c")
    sid = lax.axis_index("s")
    wid = sid * 2 + cid
    rbase = sid * RPT
    nchunk = jnp.where(sid == 15, (N - 15 * RPT) // RC, RPT // RC)

    def zrow(i, carry):
        def zcol(j, c2):
            stage[i, pl.ds(j * 16, 16)] = jnp.zeros((16,), jnp.float32)
            return c2
        return lax.fori_loop(0, D // 16, zcol, carry)
    lax.fori_loop(0, RC, zrow, 0)

    def ztab(k, carry):
        pltpu.sync_copy(stage, table.at[pl.ds(rbase + k * RC, RC)])
        return carry
    lax.fori_loop(0, nchunk, ztab, 0)
    plsc.subcore_barrier()

    base = wid * EPW
    ibufs = (i0, i1)
    fbufs = (f0, f1)
    sis = (si0, si1)
    sfs = (sf0, sf1)
    sas = (sa0, sa1)
    pltpu.async_copy(row_hbm.at[pl.ds(base, GB)], i0, si0)
    pltpu.async_copy(feat_hbm.at[pl.ds(base, GB)], f0, sf0)

    @pl.loop(0, NBS + 1, step=2)
    def _loop(g0):
        for b in range(2):
            g = g0 + b
            ib, fb, si, sf, sa = ibufs[b], fbufs[b], sis[b], sfs[b], sas[b]
            oib, ofb = ibufs[1 - b], fbufs[1 - b]
            osi, osf, osa = sis[1 - b], sfs[1 - b], sas[1 - b]

            @pl.when(g < NBS)
            def _():
                off = base + g * GB
                pltpu.make_async_copy(
                    row_hbm.at[pl.ds(off, GB)], ib, si).wait()
                pltpu.make_async_copy(
                    feat_hbm.at[pl.ds(off, GB)], fb, sf).wait()
                pltpu.async_copy(fb, table.at[ib], sa, add=True)

            @pl.when(g + 1 < NBS)
            def _():
                @pl.when(g >= 1)
                def _():
                    pltpu.make_async_copy(ofb, table.at[oib], osa).wait()
                noff = base + (g + 1) * GB
                pltpu.async_copy(row_hbm.at[pl.ds(noff, GB)], oib, osi)
                pltpu.async_copy(feat_hbm.at[pl.ds(noff, GB)], ofb, osf)

    pltpu.make_async_copy(f1, table.at[i1], sa1).wait()
    pltpu.make_async_copy(f0, table.at[i0], sa0).wait()
    plsc.subcore_barrier()

    def wb(k, carry):
        r = rbase + k * RC
        pltpu.sync_copy(table.at[pl.ds(r, RC)], stage)
        pltpu.sync_copy(stage, out_hbm.at[pl.ds(cid * N + r, RC)])
        return carry
    lax.fori_loop(0, nchunk, wb, 0)


def _bdot(a, b):
    return jnp.dot(a.astype(jnp.bfloat16), b.astype(jnp.bfloat16),
                   preferred_element_type=jnp.float32)


def _edge_body(hr, hc, s, w1r, w1c, m, b1, w2, b2, aw, ab, feat):
    z = _bdot(hr[...], w1r[...]) + _bdot(hc[...], w1c[...])
    z = z + lax.dot_general(s[...], m[...], (((0,), (0,)), ((), ())))
    z = jax.nn.silu(z + b1[...])
    mij = jax.nn.silu(_bdot(z, w2[...]) + b2[...])
    att = jax.nn.sigmoid(mij @ aw[...] + ab[...])
    feat[...] = mij * att


def _equiv_body(hr, hc, s, w1r, w1c, m, b1, w2, b2, w3, wdx, wrad, trans):
    z = _bdot(hr[...], w1r[...]) + _bdot(hc[...], w1c[...])
    z = z + lax.dot_general(s[...], m[...], (((0,), (0,)), ((), ())))
    z = jax.nn.silu(z + b1[...])
    z = jax.nn.silu(_bdot(z, w2[...]) + b2[...])
    phi = z @ w3[...]
    dx = lax.dot_general(s[...], wdx[...], (((0,), (0,)), ((), ())))
    rad = lax.dot_general(s[...], wrad[...], (((0,), (0,)), ((), ())))
    cdn = dx / (jnp.sqrt(rad + 1e-8) + 1.0)
    trans[...] = cdn * phi


def _node_body(h, a0, a1, w1h, w1a, b1, w2, b2, out):
    agg = (a0[...] + a1[...]) * (1.0 / NORM_FACTOR)
    z = jax.nn.silu(_bdot(h[...], w1h[...]) + _bdot(agg, w1a[...]) + b1[...])
    out[...] = h[...] + _bdot(z, w2[...]) + b2[...]


def _xupd_body(x, p0, p1, out):
    upd = (p0[...] + p1[...])[:, 0:3] * (1.0 / NORM_FACTOR)
    out[...] = x[...] + upd


def _hrspec():
    return pl.BlockSpec((TE, D), lambda i: (i, 0))


def _hcspec():
    return pl.BlockSpec((TE, D), lambda i: (i + E // TE, 0))


def _espec(w):
    return pl.BlockSpec((TE, w), lambda i: (i, 0))


def _wspec(r, c):
    return pl.BlockSpec((r, c), lambda i: (0, 0))


def _sspec():
    return pl.BlockSpec((5, TE), lambda i: (0, i))


def _scal_mat(w1e):
    return jnp.concatenate(
        [w1e[0:1], jnp.zeros((3, D), jnp.float32), w1e[1:2]], axis=0)


def _edge_mlp(hcat, s, p):
    return pl.pallas_call(
        _edge_body,
        grid=(E // TE,),
        in_specs=[_hrspec(), _hcspec(), _sspec(),
                  _wspec(D, D), _wspec(D, D), _wspec(5, D), _wspec(1, D),
                  _wspec(D, D), _wspec(1, D), _wspec(D, 1), _wspec(1, 1)],
        out_specs=_espec(D),
        out_shape=jax.ShapeDtypeStruct((E, D), jnp.float32),
    )(hcat, hcat, s, p['e_w1'][0:D], p['e_w1'][D:2 * D],
      _scal_mat(p['e_w1'][2 * D:]), p['e_b1'].reshape(1, D),
      p['e_w2'], p['e_b2'].reshape(1, D), p['a_w'], p['a_b'].reshape(1, 1))


def _equiv_mlp(hcat, s, p):
    wdx = jnp.zeros((5, D), jnp.float32).at[1, 0].set(1.0) \
        .at[2, 1].set(1.0).at[3, 2].set(1.0)
    wrad = jnp.zeros((5, D), jnp.float32).at[0].set(1.0)
    return pl.pallas_call(
        _equiv_body,
        grid=(E // TE,),
        in_specs=[_hrspec(), _hcspec(), _sspec(),
                  _wspec(D, D), _wspec(D, D), _wspec(5, D), _wspec(1, D),
                  _wspec(D, D), _wspec(1, D), _wspec(D, 1),
                  _wspec(5, D), _wspec(5, D)],
        out_specs=_espec(D),
        out_shape=jax.ShapeDtypeStruct((E, D), jnp.float32),
    )(hcat, hcat, s, p['c_w1'][0:D], p['c_w1'][D:2 * D],
      _scal_mat(p['c_w1'][2 * D:]), p['c_b1'].reshape(1, D),
      p['c_w2'], p['c_b2'].reshape(1, D), p['c_w3'], wdx, wrad)


def _node_mlp(h, parts, p):
    nspec = pl.BlockSpec((TN, D), lambda i: (i, 0))
    return pl.pallas_call(
        _node_body,
        grid=(N // TN,),
        in_specs=[nspec,
                  pl.BlockSpec((TN, D), lambda i: (i, 0)),
                  pl.BlockSpec((TN, D), lambda i: (i + N // TN, 0)),
                  _wspec(D, D), _wspec(D, D), _wspec(1, D),
                  _wspec(D, D), _wspec(1, D)],
        out_specs=nspec,
        out_shape=jax.ShapeDtypeStruct((N, D), jnp.float32),
    )(h, parts, parts, p['n_w1'][0:D], p['n_w1'][D:2 * D],
      p['n_b1'].reshape(1, D), p['n_w2'], p['n_b2'].reshape(1, D))


def _x_update(x, xparts):
    nspec3 = pl.BlockSpec((TN, 3), lambda i: (i, 0))
    return pl.pallas_call(
        _xupd_body,
        grid=(N // TN,),
        in_specs=[nspec3,
                  pl.BlockSpec((TN, D), lambda i: (i, 0)),
                  pl.BlockSpec((TN, D), lambda i: (i + N // TN, 0))],
        out_specs=nspec3,
        out_shape=jax.ShapeDtypeStruct((N, 3), jnp.float32),
    )(x, xparts, xparts)


def kernel(h, x, edge_attr, params, edge_index):
    row = edge_index[0]
    idxcat = edge_index.reshape(2 * E)

    hcat, flat4 = _gather_hx(h, x[:, 0], x[:, 1], x[:, 2], idxcat)
    s = jnp.concatenate(
        [flat4.reshape(4, E), edge_attr.reshape(1, E)], axis=0)

    p0 = params['gcl_0']
    feat = _edge_mlp(hcat, s, p0)
    parts = _scatter_d(feat, row)
    h = _node_mlp(h, parts, p0)

    p1 = params['gcl_1']
    hcat = _gather_h(h, idxcat)
    feat = _edge_mlp(hcat, s, p1)
    parts = _scatter_d(feat, row)
    h = _node_mlp(h, parts, p1)

    pe = params['equiv']
    hcat = _gather_h(h, idxcat)
    trans = _equiv_mlp(hcat, s, pe)
    xparts = _scatter_d(trans, row)
    x = _x_update(x, xparts)

    return (h, x)


# speedup vs baseline: 1.3067x; 1.3067x over previous
import functools

import jax
import jax.numpy as jnp
from jax import lax
from jax.experimental import pallas as pl
from jax.experimental.pallas import tpu as pltpu
from jax.experimental.pallas import tpu_sc as plsc

N = 10000
E = 320000
E2 = E // 2
D = 128
NW = 32
GB = 80
SGB = 40
SB = 1000
RPT = 640
RC = 80
TE = 3200
TN = 2000
NORM_FACTOR = 100.0

_mesh = plsc.VectorSubcoreMesh(core_axis_name="c", subcore_axis_name="s")
_sc_params = pltpu.CompilerParams(needs_layout_passes=False)


def _worker_id():
    return lax.axis_index("s") * 2 + lax.axis_index("c")


def _pipelined_gather(ne, off, h_hbm, idxcat_hbm, hcat_hbm, idx_all, bufs,
                      ibufs, sgs, sws):
    wid = _worker_id()
    cpw = 2 * ne // NW
    nbg = cpw // GB
    pos = wid * cpw
    src = jnp.where(pos < ne, off + pos, E + off + pos - ne)
    pltpu.sync_copy(idxcat_hbm.at[pl.ds(src, cpw)], idx_all)
    for j in range(2):
        pltpu.async_copy(
            h_hbm.at[idx_all.at[pl.ds(j * GB, GB)]], bufs[j], sgs[j])

    @pl.loop(0, nbg + (-nbg) % 4, step=4)
    def _loop(g0):
        for b in range(4):
            g = g0 + b
            nb = (b + 2) % 4
            wb = b % 2

            @pl.when(g < nbg)
            def _():
                pltpu.make_async_copy(
                    h_hbm.at[idx_all.at[pl.ds(g * GB, GB)]],
                    bufs[b], sgs[b]).wait()

                @pl.when(g + 2 < nbg)
                def _():
                    pltpu.async_copy(
                        h_hbm.at[idx_all.at[pl.ds((g + 2) * GB, GB)]],
                        bufs[nb], sgs[nb])

                @pl.when(g >= 2)
                def _():
                    pltpu.make_async_copy(
                        ibufs[wb],
                        hcat_hbm.at[pl.ds(pos + (g - 2) * GB, GB)],
                        sws[wb]).wait()

                def conv(r, carry):
                    for k in range(D // 32):
                        a = bufs[b][r, pl.ds(k * 32, 16)]
                        bb = bufs[b][r, pl.ds(k * 32 + 16, 16)]
                        p = plsc.pack(a, bb,
                                      format=plsc.PackFormat.INTERLEAVED)
                        ibufs[wb][r, pl.ds(k * 16, 16)] = plsc.bitcast(
                            p, jnp.int32)
                    return carry
                lax.fori_loop(0, GB, conv, 0)
                pltpu.async_copy(
                    ibufs[wb], hcat_hbm.at[pl.ds(pos + g * GB, GB)],
                    sws[wb])

    for t in range(max(0, nbg - 2), nbg):
        pltpu.make_async_copy(
            ibufs[t % 2], hcat_hbm.at[pl.ds(pos + t * GB, GB)],
            sws[t % 2]).wait()


def _make_gather(ne, off):
    @functools.partial(
        pl.kernel, mesh=_mesh,
        out_type=jax.ShapeDtypeStruct((2 * ne, D // 2), jnp.int32),
        scratch_types=[
            pltpu.VMEM((2 * ne // NW,), jnp.int32),
        ] + [pltpu.VMEM((GB, D), jnp.float32)] * 4
        + [pltpu.VMEM((GB, D // 2), jnp.int32)] * 2
        + [pltpu.SemaphoreType.DMA] * 6,
        compiler_params=_sc_params,
    )
    def _gather(h_hbm, idxcat_hbm, hcat_hbm,
                idx_all, b0, b1, b2, b3, p0, p1,
                sg0, sg1, sg2, sg3, sw0, sw1):
        _pipelined_gather(ne, off, h_hbm, idxcat_hbm, hcat_hbm, idx_all,
                          (b0, b1, b2, b3), (p0, p1),
                          (sg0, sg1, sg2, sg3), (sw0, sw1))
    return _gather


_gather_A = _make_gather(E2, 0)
_gather_B = _make_gather(E2, E2)

_EPW = E // NW


@functools.partial(
    pl.kernel, mesh=_mesh,
    out_type=[
        jax.ShapeDtypeStruct((2 * E2, D // 2), jnp.int32),
        jax.ShapeDtypeStruct((4 * E,), jnp.float32),
    ],
    scratch_types=[
        pltpu.VMEM((2 * E2 // NW,), jnp.int32),
        pltpu.VMEM((_EPW,), jnp.int32),
        pltpu.VMEM((_EPW,), jnp.int32),
    ] + [pltpu.VMEM((GB, D), jnp.float32)] * 4
    + [pltpu.VMEM((GB, D // 2), jnp.int32)] * 2 + [
        pltpu.VMEM((N,), jnp.float32),
        pltpu.VMEM((N,), jnp.float32),
        pltpu.VMEM((N,), jnp.float32),
        pltpu.VMEM((SB,), jnp.float32),
        pltpu.VMEM((SB,), jnp.float32),
        pltpu.VMEM((SB,), jnp.float32),
        pltpu.VMEM((SB,), jnp.float32),
    ] + [pltpu.SemaphoreType.DMA] * 6,
    compiler_params=_sc_params,
)
def _gather_hx_A(h_hbm, x0_hbm, x1_hbm, x2_hbm, idxcat_hbm,
                 hcat_hbm, flat4_hbm,
                 idx_all, idx_r, idx_c, b0, b1, b2, b3, p0, p1, x0, x1, x2,
                 sr, s0, s1, s2,
                 sg0, sg1, sg2, sg3, sw0, sw1):
    wid = _worker_id()
    pbase = wid * _EPW
    pltpu.sync_copy(idxcat_hbm.at[pl.ds(pbase, _EPW)], idx_r)
    pltpu.sync_copy(idxcat_hbm.at[pl.ds(E + pbase, _EPW)], idx_c)
    pltpu.sync_copy(x0_hbm, x0)
    pltpu.sync_copy(x1_hbm, x1)
    pltpu.sync_copy(x2_hbm, x2)

    def chunk(c, carry):
        def group(k, c2):
            src = pl.ds(c * SB + k * 16, 16)
            dst = pl.ds(k * 16, 16)
            ir = idx_r[src]
            ic = idx_c[src]
            d0 = plsc.load_gather(x0, [ir]) - plsc.load_gather(x0, [ic])
            d1 = plsc.load_gather(x1, [ir]) - plsc.load_gather(x1, [ic])
            d2 = plsc.load_gather(x2, [ir]) - plsc.load_gather(x2, [ic])
            sr[dst] = d0 * d0 + d1 * d1 + d2 * d2
            s0[dst] = d0
            s1[dst] = d1
            s2[dst] = d2
            return c2
        lax.fori_loop(0, SB // 16, group, carry)
        off = pbase + c * SB
        pltpu.sync_copy(sr, flat4_hbm.at[pl.ds(0 * E + off, SB)])
        pltpu.sync_copy(s0, flat4_hbm.at[pl.ds(1 * E + off, SB)])
        pltpu.sync_copy(s1, flat4_hbm.at[pl.ds(2 * E + off, SB)])
        pltpu.sync_copy(s2, flat4_hbm.at[pl.ds(3 * E + off, SB)])
        return carry
    lax.fori_loop(0, _EPW // SB, chunk, 0)

    _pipelined_gather(E2, 0, h_hbm, idxcat_hbm, hcat_hbm, idx_all,
                      (b0, b1, b2, b3), (p0, p1),
                      (sg0, sg1, sg2, sg3), (sw0, sw1))


def _make_scatter(ne, off):
    epw = ne // NW
    nbs = epw // SGB

    @functools.partial(
        pl.kernel, mesh=_mesh,
        out_type=jax.ShapeDtypeStruct((2 * N, D), jnp.float32),
        scratch_types=[pltpu.VMEM((SGB,), jnp.int32)] * 4
        + [pltpu.VMEM((SGB, D), jnp.float32)] * 4 + [
            pltpu.VMEM((RC, D), jnp.float32),
            pltpu.VMEM_SHARED((N, D), jnp.float32),
        ] + [pltpu.SemaphoreType.DMA] * 12,
        compiler_params=_sc_params,
    )
    def _scatter(feat_hbm, row_hbm, out_hbm,
                 i0, i1, i2, i3, f0, f1, f2, f3, stage, table,
                 si0, si1, si2, si3, sf0, sf1, sf2, sf3,
                 sa0, sa1, sa2, sa3):
        cid = lax.axis_index("c")
        sid = lax.axis_index("s")
        wid = sid * 2 + cid
        rbase = sid * RPT
        nchunk = jnp.where(sid == 15, (N - 15 * RPT) // RC, RPT // RC)

        def zrow(i, carry):
            def zcol(j, c2):
                stage[i, pl.ds(j * 16, 16)] = jnp.zeros((16,), jnp.float32)
                return c2
            return lax.fori_loop(0, D // 16, zcol, carry)
        lax.fori_loop(0, RC, zrow, 0)

        def ztab(k, carry):
            pltpu.sync_copy(stage, table.at[pl.ds(rbase + k * RC, RC)])
            return carry
        lax.fori_loop(0, nchunk, ztab, 0)
        plsc.subcore_barrier()

        base = off + wid * epw
        fbase = wid * epw
        ibufs = (i0, i1, i2, i3)
        fbufs = (f0, f1, f2, f3)
        sis = (si0, si1, si2, si3)
        sfs = (sf0, sf1, sf2, sf3)
        sas = (sa0, sa1, sa2, sa3)
        for j in range(2):
            pltpu.async_copy(
                row_hbm.at[pl.ds(base + j * SGB, SGB)], ibufs[j], sis[j])
            pltpu.async_copy(
                feat_hbm.at[pl.ds(fbase + j * SGB, SGB)], fbufs[j], sfs[j])

        @pl.loop(0, nbs + (-nbs) % 4, step=4)
        def _loop(g0):
            for b in range(4):
                g = g0 + b
                nb = (b + 2) % 4

                @pl.when(g < nbs)
                def _():
                    pltpu.make_async_copy(
                        row_hbm.at[pl.ds(base + g * SGB, SGB)],
                        ibufs[b], sis[b]).wait()
                    pltpu.make_async_copy(
                        feat_hbm.at[pl.ds(fbase + g * SGB, SGB)],
                        fbufs[b], sfs[b]).wait()
                    pltpu.async_copy(fbufs[b], table.at[ibufs[b]],
                                     sas[b], add=True)

                    @pl.when(g + 2 < nbs)
                    def _():
                        @pl.when(g >= 2)
                        def _():
                            pltpu.make_async_copy(
                                fbufs[nb], table.at[ibufs[nb]],
                                sas[nb]).wait()
                        pltpu.async_copy(
                            row_hbm.at[pl.ds(base + (g + 2) * SGB, SGB)],
                            ibufs[nb], sis[nb])
                        pltpu.async_copy(
                            feat_hbm.at[pl.ds(fbase + (g + 2) * SGB, SGB)],
                            fbufs[nb], sfs[nb])

        for t in range(max(0, nbs - 4), nbs):
            pltpu.make_async_copy(
                fbufs[t % 4], table.at[ibufs[t % 4]], sas[t % 4]).wait()
        plsc.subcore_barrier()

        def wb(k, carry):
            r = rbase + k * RC
            pltpu.sync_copy(table.at[pl.ds(r, RC)], stage)
            pltpu.sync_copy(stage, out_hbm.at[pl.ds(cid * N + r, RC)])
            return carry
        lax.fori_loop(0, nchunk, wb, 0)

    return _scatter


_scatter_A = _make_scatter(E2, 0)
_scatter_B = _make_scatter(E2, E2)


def _bdot(a, b):
    return jnp.dot(a.astype(jnp.bfloat16), b.astype(jnp.bfloat16),
                   preferred_element_type=jnp.float32)


def _unpack(v):
    a = lax.bitcast_convert_type(v << 16, jnp.float32)
    b = lax.bitcast_convert_type(v & jnp.int32(-65536), jnp.float32)
    return a, b


_PA = [32 * (m // 16) + m % 16 for m in range(D // 2)]
_PB = [p + 16 for p in _PA]


def _hdot(packed, wa, wb):
    a, b = _unpack(packed)
    return _bdot(a, wa) + _bdot(b, wb)


def _edge_body(hr, hc, s, w1ra, w1rb, w1ca, w1cb, m, b1, w2, b2, aw, ab,
               feat):
    z = _hdot(hr[...], w1ra[...], w1rb[...]) \
        + _hdot(hc[...], w1ca[...], w1cb[...])
    z = z + lax.dot_general(s[...], m[...], (((0,), (0,)), ((), ())))
    z = jax.nn.silu(z + b1[...])
    mij = jax.nn.silu(_bdot(z, w2[...]) + b2[...])
    att = jax.nn.sigmoid(mij @ aw[...] + ab[...])
    feat[...] = mij * att


def _equiv_body(hr, hc, s, w1ra, w1rb, w1ca, w1cb, m, b1, w2, b2, w3, wdx,
                wrad, trans):
    z = _hdot(hr[...], w1ra[...], w1rb[...]) \
        + _hdot(hc[...], w1ca[...], w1cb[...])
    z = z + lax.dot_general(s[...], m[...], (((0,), (0,)), ((), ())))
    z = jax.nn.silu(z + b1[...])
    z = jax.nn.silu(_bdot(z, w2[...]) + b2[...])
    phi = z @ w3[...]
    dx = lax.dot_general(s[...], wdx[...], (((0,), (0,)), ((), ())))
    rad = lax.dot_general(s[...], wrad[...], (((0,), (0,)), ((), ())))
    cdn = dx / (jnp.sqrt(rad + 1e-8) + 1.0)
    trans[...] = cdn * phi


def _node_body(h, a0, a1, a2, a3, w1h, w1a, b1, w2, b2, out):
    agg = (a0[...] + a1[...] + a2[...] + a3[...]) * (1.0 / NORM_FACTOR)
    z = jax.nn.silu(_bdot(h[...], w1h[...]) + _bdot(agg, w1a[...]) + b1[...])
    out[...] = h[...] + _bdot(z, w2[...]) + b2[...]


def _xupd_body(x, p0, p1, p2, p3, out):
    upd = (p0[...] + p1[...] + p2[...] + p3[...])[:, 0:3] \
        * (1.0 / NORM_FACTOR)
    out[...] = x[...] + upd


def _hrspec():
    return pl.BlockSpec((TE, D // 2), lambda i: (i, 0))


def _hcspec():
    return pl.BlockSpec((TE, D // 2), lambda i: (i + E2 // TE, 0))


def _espec(w):
    return pl.BlockSpec((TE, w), lambda i: (i, 0))


def _wspec(r, c):
    return pl.BlockSpec((r, c), lambda i: (0, 0))


def _sspec(s_off):
    so = s_off // TE
    return pl.BlockSpec((5, TE), lambda i: (0, i + so))


def _scal_mat(w1e):
    return jnp.concatenate(
        [w1e[0:1], jnp.zeros((3, D), jnp.float32), w1e[1:2]], axis=0)


def _edge_mlp(hcat, s, p, s_off):
    pa = jnp.array(_PA, jnp.int32)
    pb = jnp.array(_PB, jnp.int32)
    w1r = p['e_w1'][0:D]
    w1c = p['e_w1'][D:2 * D]
    return pl.pallas_call(
        _edge_body,
        grid=(E2 // TE,),
        in_specs=[_hrspec(), _hcspec(), _sspec(s_off)]
        + [_wspec(D // 2, D)] * 4
        + [_wspec(5, D), _wspec(1, D),
           _wspec(D, D), _wspec(1, D), _wspec(D, 1), _wspec(1, 1)],
        out_specs=_espec(D),
        out_shape=jax.ShapeDtypeStruct((E2, D), jnp.float32),
    )(hcat, hcat, s, w1r[pa], w1r[pb], w1c[pa], w1c[pb],
      _scal_mat(p['e_w1'][2 * D:]), p['e_b1'].reshape(1, D),
      p['e_w2'], p['e_b2'].reshape(1, D), p['a_w'], p['a_b'].reshape(1, 1))


def _equiv_mlp(hcat, s, p, s_off):
    pa = jnp.array(_PA, jnp.int32)
    pb = jnp.array(_PB, jnp.int32)
    w1r = p['c_w1'][0:D]
    w1c = p['c_w1'][D:2 * D]
    wdx = jnp.zeros((5, D), jnp.float32).at[1, 0].set(1.0) \
        .at[2, 1].set(1.0).at[3, 2].set(1.0)
    wrad = jnp.zeros((5, D), jnp.float32).at[0].set(1.0)
    return pl.pallas_call(
        _equiv_body,
        grid=(E2 // TE,),
        in_specs=[_hrspec(), _hcspec(), _sspec(s_off)]
        + [_wspec(D // 2, D)] * 4
        + [_wspec(5, D), _wspec(1, D),
           _wspec(D, D), _wspec(1, D), _wspec(D, 1),
           _wspec(5, D), _wspec(5, D)],
        out_specs=_espec(D),
        out_shape=jax.ShapeDtypeStruct((E2, D), jnp.float32),
    )(hcat, hcat, s, w1r[pa], w1r[pb], w1c[pa], w1c[pb],
      _scal_mat(p['c_w1'][2 * D:]), p['c_b1'].reshape(1, D),
      p['c_w2'], p['c_b2'].reshape(1, D), p['c_w3'], wdx, wrad)


def _pspecs():
    return [pl.BlockSpec((TN, D), lambda i: (i, 0)),
            pl.BlockSpec((TN, D), lambda i: (i + N // TN, 0))]


def _node_mlp(h, partsA, partsB, p):
    nspec = pl.BlockSpec((TN, D), lambda i: (i, 0))
    return pl.pallas_call(
        _node_body,
        grid=(N // TN,),
        in_specs=[nspec] + _pspecs() + _pspecs()
        + [_wspec(D, D), _wspec(D, D), _wspec(1, D),
           _wspec(D, D), _wspec(1, D)],
        out_specs=nspec,
        out_shape=jax.ShapeDtypeStruct((N, D), jnp.float32),
    )(h, partsA, partsA, partsB, partsB,
      p['n_w1'][0:D], p['n_w1'][D:2 * D],
      p['n_b1'].reshape(1, D), p['n_w2'], p['n_b2'].reshape(1, D))


def _x_update(x, xpartsA, xpartsB):
    nspec3 = pl.BlockSpec((TN, 3), lambda i: (i, 0))
    return pl.pallas_call(
        _xupd_body,
        grid=(N // TN,),
        in_specs=[nspec3] + _pspecs() + _pspecs(),
        out_specs=nspec3,
        out_shape=jax.ShapeDtypeStruct((N, 3), jnp.float32),
    )(x, xpartsA, xpartsA, xpartsB, xpartsB)


def kernel(h, x, edge_attr, params, edge_index):
    row = edge_index[0]
    idxcat = edge_index.reshape(2 * E)

    hcatA, flat4 = _gather_hx_A(h, x[:, 0], x[:, 1], x[:, 2], idxcat)
    hcatB = _gather_B(h, idxcat)
    s = jnp.concatenate(
        [flat4.reshape(4, E), edge_attr.reshape(1, E)], axis=0)

    p0 = params['gcl_0']
    featA = _edge_mlp(hcatA, s, p0, 0)
    featB = _edge_mlp(hcatB, s, p0, E2)
    partsA = _scatter_A(featA, row)
    partsB = _scatter_B(featB, row)
    h = _node_mlp(h, partsA, partsB, p0)

    p1 = params['gcl_1']
    hcatA = _gather_A(h, idxcat)
    hcatB = _gather_B(h, idxcat)
    featA = _edge_mlp(hcatA, s, p1, 0)
    featB = _edge_mlp(hcatB, s, p1, E2)
    partsA = _scatter_A(featA, row)
    partsB = _scatter_B(featB, row)
    h = _node_mlp(h, partsA, partsB, p1)

    pe = params['equiv']
    hcatA = _gather_A(h, idxcat)
    hcatB = _gather_B(h, idxcat)
    transA = _equiv_mlp(hcatA, s, pe, 0)
    transB = _equiv_mlp(hcatB, s, pe, E2)
    xpartsA = _scatter_A(transA, row)
    xpartsB = _scatter_B(transB, row)
    x = _x_update(x, xpartsA, xpartsB)

    return (h, x)


# speedup vs baseline: 1.4029x; 1.0736x over previous
import functools

import jax
import jax.numpy as jnp
from jax import lax
from jax.experimental import pallas as pl
from jax.experimental.pallas import tpu as pltpu
from jax.experimental.pallas import tpu_sc as plsc

N = 10000
E = 320000
E2 = E // 2
D = 128
NW = 32
GB = 80
SGB = 40
SB = 2000
RPT = 640
RC = 80
TE = 3200
TN = 2000
NORM_FACTOR = 100.0

_mesh = plsc.VectorSubcoreMesh(core_axis_name="c", subcore_axis_name="s")
_sc_params = pltpu.CompilerParams(needs_layout_passes=False)


def _worker_id():
    return lax.axis_index("s") * 2 + lax.axis_index("c")


def _pipelined_gather(ne, off, h_hbm, idxcat_hbm, hcat_hbm, idx_all, bufs,
                      sgs, sws):
    wid = _worker_id()
    cpw = 2 * ne // NW
    nbg = cpw // GB
    pos = wid * cpw
    src = jnp.where(pos < ne, off + pos, E + off + pos - ne)
    pltpu.sync_copy(idxcat_hbm.at[pl.ds(src, cpw)], idx_all)
    for j in range(2):
        pltpu.async_copy(
            h_hbm.at[idx_all.at[pl.ds(j * GB, GB)]], bufs[j], sgs[j])

    @pl.loop(0, nbg + (-nbg) % 4, step=4)
    def _loop(g0):
        for b in range(4):
            g = g0 + b
            nb = (b + 2) % 4

            @pl.when(g < nbg)
            def _():
                pltpu.make_async_copy(
                    h_hbm.at[idx_all.at[pl.ds(g * GB, GB)]],
                    bufs[b], sgs[b]).wait()
                pltpu.async_copy(
                    bufs[b], hcat_hbm.at[pl.ds(pos + g * GB, GB)], sws[b])

                @pl.when(g + 2 < nbg)
                def _():
                    @pl.when(g >= 2)
                    def _():
                        pltpu.make_async_copy(
                            bufs[nb],
                            hcat_hbm.at[pl.ds(pos + (g - 2) * GB, GB)],
                            sws[nb]).wait()
                    pltpu.async_copy(
                        h_hbm.at[idx_all.at[pl.ds((g + 2) * GB, GB)]],
                        bufs[nb], sgs[nb])

    for t in range(max(0, nbg - 4), nbg):
        pltpu.make_async_copy(
            bufs[t % 4], hcat_hbm.at[pl.ds(pos + t * GB, GB)],
            sws[t % 4]).wait()


def _make_gather(ne, off):
    @functools.partial(
        pl.kernel, mesh=_mesh,
        out_type=jax.ShapeDtypeStruct((2 * ne, D), jnp.float32),
        scratch_types=[
            pltpu.VMEM((2 * ne // NW,), jnp.int32),
        ] + [pltpu.VMEM((GB, D), jnp.float32)] * 4
        + [pltpu.SemaphoreType.DMA] * 8,
        compiler_params=_sc_params,
    )
    def _gather(h_hbm, idxcat_hbm, hcat_hbm,
                idx_all, b0, b1, b2, b3,
                sg0, sg1, sg2, sg3, sw0, sw1, sw2, sw3):
        _pipelined_gather(ne, off, h_hbm, idxcat_hbm, hcat_hbm, idx_all,
                          (b0, b1, b2, b3), (sg0, sg1, sg2, sg3),
                          (sw0, sw1, sw2, sw3))
    return _gather


_gather_A = _make_gather(E2, 0)
_gather_B = _make_gather(E2, E2)

_EPW = E // NW


@functools.partial(
    pl.kernel, mesh=_mesh,
    out_type=[
        jax.ShapeDtypeStruct((2 * E2, D), jnp.float32),
        jax.ShapeDtypeStruct((4 * E,), jnp.float32),
    ],
    scratch_types=[
        pltpu.VMEM((2 * E2 // NW,), jnp.int32),
        pltpu.VMEM((_EPW,), jnp.int32),
        pltpu.VMEM((_EPW,), jnp.int32),
    ] + [pltpu.VMEM((GB, D), jnp.float32)] * 4 + [
        pltpu.VMEM((N,), jnp.float32),
        pltpu.VMEM((N,), jnp.float32),
        pltpu.VMEM((N,), jnp.float32),
        pltpu.VMEM((SB,), jnp.float32),
        pltpu.VMEM((SB,), jnp.float32),
        pltpu.VMEM((SB,), jnp.float32),
        pltpu.VMEM((SB,), jnp.float32),
    ] + [pltpu.SemaphoreType.DMA] * 8,
    compiler_params=_sc_params,
)
def _gather_hx_A(h_hbm, x0_hbm, x1_hbm, x2_hbm, idxcat_hbm,
                 hcat_hbm, flat4_hbm,
                 idx_all, idx_r, idx_c, b0, b1, b2, b3, x0, x1, x2,
                 sr, s0, s1, s2,
                 sg0, sg1, sg2, sg3, sw0, sw1, sw2, sw3):
    wid = _worker_id()
    pbase = wid * _EPW
    pltpu.sync_copy(idxcat_hbm.at[pl.ds(pbase, _EPW)], idx_r)
    pltpu.sync_copy(idxcat_hbm.at[pl.ds(E + pbase, _EPW)], idx_c)
    pltpu.sync_copy(x0_hbm, x0)
    pltpu.sync_copy(x1_hbm, x1)
    pltpu.sync_copy(x2_hbm, x2)

    def chunk(c, carry):
        def group(k, c2):
            src = pl.ds(c * SB + k * 16, 16)
            dst = pl.ds(k * 16, 16)
            ir = idx_r[src]
            ic = idx_c[src]
            d0 = plsc.load_gather(x0, [ir]) - plsc.load_gather(x0, [ic])
            d1 = plsc.load_gather(x1, [ir]) - plsc.load_gather(x1, [ic])
            d2 = plsc.load_gather(x2, [ir]) - plsc.load_gather(x2, [ic])
            sr[dst] = d0 * d0 + d1 * d1 + d2 * d2
            s0[dst] = d0
            s1[dst] = d1
            s2[dst] = d2
            return c2
        lax.fori_loop(0, SB // 16, group, carry)
        off = pbase + c * SB
        pltpu.sync_copy(sr, flat4_hbm.at[pl.ds(0 * E + off, SB)])
        pltpu.sync_copy(s0, flat4_hbm.at[pl.ds(1 * E + off, SB)])
        pltpu.sync_copy(s1, flat4_hbm.at[pl.ds(2 * E + off, SB)])
        pltpu.sync_copy(s2, flat4_hbm.at[pl.ds(3 * E + off, SB)])
        return carry
    lax.fori_loop(0, _EPW // SB, chunk, 0)

    _pipelined_gather(E2, 0, h_hbm, idxcat_hbm, hcat_hbm, idx_all,
                      (b0, b1, b2, b3), (sg0, sg1, sg2, sg3),
                      (sw0, sw1, sw2, sw3))


def _make_scatter(ne, off):
    epw = ne // NW
    nbs = epw // SGB

    @functools.partial(
        pl.kernel, mesh=_mesh,
        out_type=jax.ShapeDtypeStruct((2 * N, D), jnp.float32),
        scratch_types=[pltpu.VMEM((SGB,), jnp.int32)] * 4
        + [pltpu.VMEM((SGB, D), jnp.float32)] * 4 + [
            pltpu.VMEM((RC, D), jnp.float32),
            pltpu.VMEM_SHARED((N, D), jnp.float32),
        ] + [pltpu.SemaphoreType.DMA] * 12,
        compiler_params=_sc_params,
    )
    def _scatter(feat_hbm, row_hbm, out_hbm,
                 i0, i1, i2, i3, f0, f1, f2, f3, stage, table,
                 si0, si1, si2, si3, sf0, sf1, sf2, sf3,
                 sa0, sa1, sa2, sa3):
        cid = lax.axis_index("c")
        sid = lax.axis_index("s")
        wid = sid * 2 + cid
        rbase = sid * RPT
        nchunk = jnp.where(sid == 15, (N - 15 * RPT) // RC, RPT // RC)

        base = off + wid * epw
        fbase = wid * epw
        ibufs = (i0, i1, i2, i3)
        fbufs = (f0, f1, f2, f3)
        sis = (si0, si1, si2, si3)
        sfs = (sf0, sf1, sf2, sf3)
        sas = (sa0, sa1, sa2, sa3)
        for j in range(4):
            pltpu.async_copy(
                row_hbm.at[pl.ds(base + j * SGB, SGB)], ibufs[j], sis[j])
            pltpu.async_copy(
                feat_hbm.at[pl.ds(fbase + j * SGB, SGB)], fbufs[j], sfs[j])

        def zrow(i, carry):
            def zcol(j, c2):
                stage[i, pl.ds(j * 16, 16)] = jnp.zeros((16,), jnp.float32)
                return c2
            return lax.fori_loop(0, D // 16, zcol, carry)
        lax.fori_loop(0, RC, zrow, 0)

        def ztab(k, carry):
            pltpu.sync_copy(stage, table.at[pl.ds(rbase + k * RC, RC)])
            return carry
        lax.fori_loop(0, nchunk, ztab, 0)
        plsc.subcore_barrier()

        @pl.loop(0, nbs + (-nbs) % 4, step=4)
        def _loop(g0):
            for b in range(4):
                g = g0 + b
                nb = (b + 2) % 4

                @pl.when(g < nbs)
                def _():
                    pltpu.make_async_copy(
                        row_hbm.at[pl.ds(base + g * SGB, SGB)],
                        ibufs[b], sis[b]).wait()
                    pltpu.make_async_copy(
                        feat_hbm.at[pl.ds(fbase + g * SGB, SGB)],
                        fbufs[b], sfs[b]).wait()
                    pltpu.async_copy(fbufs[b], table.at[ibufs[b]],
                                     sas[b], add=True)

                    @pl.when(jnp.logical_and(g + 2 < nbs, g >= 2))
                    def _():
                        pltpu.make_async_copy(
                            fbufs[nb], table.at[ibufs[nb]],
                            sas[nb]).wait()
                        pltpu.async_copy(
                            row_hbm.at[pl.ds(base + (g + 2) * SGB, SGB)],
                            ibufs[nb], sis[nb])
                        pltpu.async_copy(
                            feat_hbm.at[pl.ds(fbase + (g + 2) * SGB, SGB)],
                            fbufs[nb], sfs[nb])

        for t in range(max(0, nbs - 4), nbs):
            pltpu.make_async_copy(
                fbufs[t % 4], table.at[ibufs[t % 4]], sas[t % 4]).wait()
        plsc.subcore_barrier()

        def wb(k, carry):
            r = rbase + k * RC
            pltpu.sync_copy(table.at[pl.ds(r, RC)], stage)
            pltpu.sync_copy(stage, out_hbm.at[pl.ds(cid * N + r, RC)])
            return carry
        lax.fori_loop(0, nchunk, wb, 0)

    return _scatter


_scatter_A = _make_scatter(E2, 0)
_scatter_B = _make_scatter(E2, E2)


def _bdot(a, b):
    return jnp.dot(a.astype(jnp.bfloat16), b.astype(jnp.bfloat16),
                   preferred_element_type=jnp.float32)


def _edge_body(hr, hc, s, w1r, w1c, m, b1, w2, b2, aw, ab, feat):
    z = _bdot(hr[...], w1r[...]) + _bdot(hc[...], w1c[...])
    z = z + lax.dot_general(s[...], m[...], (((0,), (0,)), ((), ())))
    z = jax.nn.silu(z + b1[...])
    mij = jax.nn.silu(_bdot(z, w2[...]) + b2[...])
    att = jax.nn.sigmoid(mij @ aw[...] + ab[...])
    feat[...] = mij * att


def _equiv_body(hr, hc, s, w1r, w1c, m, b1, w2, b2, w3, wdx, wrad, trans):
    z = _bdot(hr[...], w1r[...]) + _bdot(hc[...], w1c[...])
    z = z + lax.dot_general(s[...], m[...], (((0,), (0,)), ((), ())))
    z = jax.nn.silu(z + b1[...])
    z = jax.nn.silu(_bdot(z, w2[...]) + b2[...])
    phi = z @ w3[...]
    dx = lax.dot_general(s[...], wdx[...], (((0,), (0,)), ((), ())))
    rad = lax.dot_general(s[...], wrad[...], (((0,), (0,)), ((), ())))
    cdn = dx / (jnp.sqrt(rad + 1e-8) + 1.0)
    trans[...] = cdn * phi


def _node_body(h, a0, a1, a2, a3, w1h, w1a, b1, w2, b2, out):
    agg = (a0[...] + a1[...] + a2[...] + a3[...]) * (1.0 / NORM_FACTOR)
    z = jax.nn.silu(_bdot(h[...], w1h[...]) + _bdot(agg, w1a[...]) + b1[...])
    out[...] = h[...] + _bdot(z, w2[...]) + b2[...]


def _xupd_body(x, p0, p1, p2, p3, out):
    upd = (p0[...] + p1[...] + p2[...] + p3[...])[:, 0:3] \
        * (1.0 / NORM_FACTOR)
    out[...] = x[...] + upd


def _hrspec():
    return pl.BlockSpec((TE, D), lambda i: (i, 0))


def _hcspec():
    return pl.BlockSpec((TE, D), lambda i: (i + E2 // TE, 0))


def _espec(w):
    return pl.BlockSpec((TE, w), lambda i: (i, 0))


def _wspec(r, c):
    return pl.BlockSpec((r, c), lambda i: (0, 0))


def _sspec(s_off):
    so = s_off // TE
    return pl.BlockSpec((5, TE), lambda i: (0, i + so))


def _scal_mat(w1e):
    return jnp.concatenate(
        [w1e[0:1], jnp.zeros((3, D), jnp.float32), w1e[1:2]], axis=0)


def _edge_mlp(hcat, s, p, s_off):
    return pl.pallas_call(
        _edge_body,
        grid=(E2 // TE,),
        in_specs=[_hrspec(), _hcspec(), _sspec(s_off),
                  _wspec(D, D), _wspec(D, D), _wspec(5, D), _wspec(1, D),
                  _wspec(D, D), _wspec(1, D), _wspec(D, 1), _wspec(1, 1)],
        out_specs=_espec(D),
        out_shape=jax.ShapeDtypeStruct((E2, D), jnp.float32),
    )(hcat, hcat, s, p['e_w1'][0:D], p['e_w1'][D:2 * D],
      _scal_mat(p['e_w1'][2 * D:]), p['e_b1'].reshape(1, D),
      p['e_w2'], p['e_b2'].reshape(1, D), p['a_w'], p['a_b'].reshape(1, 1))


def _equiv_mlp(hcat, s, p, s_off):
    wdx = jnp.zeros((5, D), jnp.float32).at[1, 0].set(1.0) \
        .at[2, 1].set(1.0).at[3, 2].set(1.0)
    wrad = jnp.zeros((5, D), jnp.float32).at[0].set(1.0)
    return pl.pallas_call(
        _equiv_body,
        grid=(E2 // TE,),
        in_specs=[_hrspec(), _hcspec(), _sspec(s_off),
                  _wspec(D, D), _wspec(D, D), _wspec(5, D), _wspec(1, D),
                  _wspec(D, D), _wspec(1, D), _wspec(D, 1),
                  _wspec(5, D), _wspec(5, D)],
        out_specs=_espec(D),
        out_shape=jax.ShapeDtypeStruct((E2, D), jnp.float32),
    )(hcat, hcat, s, p['c_w1'][0:D], p['c_w1'][D:2 * D],
      _scal_mat(p['c_w1'][2 * D:]), p['c_b1'].reshape(1, D),
      p['c_w2'], p['c_b2'].reshape(1, D), p['c_w3'], wdx, wrad)


def _pspecs():
    return [pl.BlockSpec((TN, D), lambda i: (i, 0)),
            pl.BlockSpec((TN, D), lambda i: (i + N // TN, 0))]


def _node_mlp(h, partsA, partsB, p):
    nspec = pl.BlockSpec((TN, D), lambda i: (i, 0))
    return pl.pallas_call(
        _node_body,
        grid=(N // TN,),
        in_specs=[nspec] + _pspecs() + _pspecs()
        + [_wspec(D, D), _wspec(D, D), _wspec(1, D),
           _wspec(D, D), _wspec(1, D)],
        out_specs=nspec,
        out_shape=jax.ShapeDtypeStruct((N, D), jnp.float32),
    )(h, partsA, partsA, partsB, partsB,
      p['n_w1'][0:D], p['n_w1'][D:2 * D],
      p['n_b1'].reshape(1, D), p['n_w2'], p['n_b2'].reshape(1, D))


def _x_update(x, xpartsA, xpartsB):
    nspec3 = pl.BlockSpec((TN, 3), lambda i: (i, 0))
    return pl.pallas_call(
        _xupd_body,
        grid=(N // TN,),
        in_specs=[nspec3] + _pspecs() + _pspecs(),
        out_specs=nspec3,
        out_shape=jax.ShapeDtypeStruct((N, 3), jnp.float32),
    )(x, xpartsA, xpartsA, xpartsB, xpartsB)


def kernel(h, x, edge_attr, params, edge_index):
    row = edge_index[0]
    idxcat = edge_index.reshape(2 * E)

    hcatA, flat4 = _gather_hx_A(h, x[:, 0], x[:, 1], x[:, 2], idxcat)
    hcatB = _gather_B(h, idxcat)
    s = jnp.concatenate(
        [flat4.reshape(4, E), edge_attr.reshape(1, E)], axis=0)

    p0 = params['gcl_0']
    featA = _edge_mlp(hcatA, s, p0, 0)
    featB = _edge_mlp(hcatB, s, p0, E2)
    partsA = _scatter_A(featA, row)
    partsB = _scatter_B(featB, row)
    h = _node_mlp(h, partsA, partsB, p0)

    p1 = params['gcl_1']
    hcatA = _gather_A(h, idxcat)
    hcatB = _gather_B(h, idxcat)
    featA = _edge_mlp(hcatA, s, p1, 0)
    featB = _edge_mlp(hcatB, s, p1, E2)
    partsA = _scatter_A(featA, row)
    partsB = _scatter_B(featB, row)
    h = _node_mlp(h, partsA, partsB, p1)

    pe = params['equiv']
    hcatA = _gather_A(h, idxcat)
    hcatB = _gather_B(h, idxcat)
    transA = _equiv_mlp(hcatA, s, pe, 0)
    transB = _equiv_mlp(hcatB, s, pe, E2)
    xpartsA = _scatter_A(transA, row)
    xpartsB = _scatter_B(transB, row)
    x = _x_update(x, xpartsA, xpartsB)

    return (h, x)
